# baseline (speedup 1.0000x reference)
import dataclasses
import functools

import jax
import jax.numpy as jnp
from jax import lax
from jax.experimental import pallas as pl
from jax.experimental.pallas import tpu as pltpu
from jax.experimental.pallas import tpu_sc as plsc

O = 10000
T = 160000
DIN = 128
H = 128
DOUT = 128

NC = 2
NS = 16
NW = NC * NS

CH = 128
NCHUNKS = T // CH
RPT = 624
RPT_TAIL = O - NS * RPT
CW = 16

_f32 = jnp.float32


def _worker_id():
    return lax.axis_index("s") * NC + lax.axis_index("c")


def _sc_gather(obj_vecs, s_idx, o_idx):
    mesh = plsc.VectorSubcoreMesh(core_axis_name="c", subcore_axis_name="s")
    out_type = (jax.ShapeDtypeStruct((T, DIN), _f32),
                jax.ShapeDtypeStruct((T, DIN), _f32))

    @functools.partial(
        pl.kernel, mesh=mesh, out_type=out_type,
        scratch_types=[
            pltpu.VMEM((CH,), jnp.int32),
            pltpu.VMEM((CH,), jnp.int32),
            pltpu.VMEM((CH, DIN), _f32),
            pltpu.VMEM((CH, DIN), _f32),
            pltpu.SemaphoreType.DMA,
            pltpu.SemaphoreType.DMA,
        ],
    )
    def k(obj_hbm, sidx_hbm, oidx_hbm, sg_hbm, og_hbm,
          sidx_v, oidx_v, srows_v, orows_v, sem_s, sem_o):
        wid = _worker_id()
        nch = 39 + jnp.where(wid < (NCHUNKS - 39 * NW), 1, 0)

        @pl.loop(0, nch)
        def _(kk):
            cbase = (wid + NW * kk) * CH
            pltpu.sync_copy(sidx_hbm.at[pl.ds(cbase, CH)], sidx_v)
            pltpu.sync_copy(oidx_hbm.at[pl.ds(cbase, CH)], oidx_v)
            cs = pltpu.async_copy(obj_hbm.at[sidx_v], srows_v, sem_s)
            co = pltpu.async_copy(obj_hbm.at[oidx_v], orows_v, sem_o)
            cs.wait()
            co.wait()
            pltpu.sync_copy(srows_v, sg_hbm.at[pl.ds(cbase, CH)])
            pltpu.sync_copy(orows_v, og_hbm.at[pl.ds(cbase, CH)])

    return k(obj_vecs, s_idx, o_idx)


def _sc_scatter(s_out, o_out, s_idx, o_idx, zeros_p):
    mesh = plsc.VectorSubcoreMesh(core_axis_name="c", subcore_axis_name="s")
    out_type = (jax.ShapeDtypeStruct((NC, O, H), _f32),
                jax.ShapeDtypeStruct((NW, O), _f32))
    cp = pltpu.CompilerParams()
    if "needs_layout_passes" in pltpu.CompilerParams.__dataclass_fields__:
        cp = dataclasses.replace(cp, needs_layout_passes=False)

    @functools.partial(
        pl.kernel, mesh=mesh, out_type=out_type, compiler_params=cp,
        scratch_types=[
            pltpu.VMEM((CH,), jnp.int32),
            pltpu.VMEM((CH, H), _f32),
            pltpu.VMEM((O,), _f32),
            pltpu.VMEM_SHARED((O, H), _f32),
        ],
    )
    def k(sout_hbm, oout_hbm, sidx_hbm, oidx_hbm, zp_hbm,
          pooled_out, counts_out,
          idx_v, rows_v, cnt_v, pooled_sh):
        cid = lax.axis_index("c")
        sid = lax.axis_index("s")
        wid = sid * NC + cid
        base = sid * RPT
        zeros16 = jnp.zeros((16,), _f32)
        ones16 = jnp.ones((16,), _f32)

        @pl.loop(0, O, step=16)
        def _(i):
            cnt_v[pl.ds(i, 16)] = zeros16
        pltpu.sync_copy(zp_hbm, rows_v)
        for j in range(4):
            pltpu.sync_copy(rows_v, pooled_sh.at[pl.ds(base + j * CH, CH)])
        pltpu.sync_copy(rows_v.at[pl.ds(0, RPT - 4 * CH)],
                        pooled_sh.at[pl.ds(base + 4 * CH, RPT - 4 * CH)])

        @pl.when(sid == NS - 1)
        def _():
            pltpu.sync_copy(rows_v.at[pl.ds(0, RPT_TAIL)],
                            pooled_sh.at[pl.ds(NS * RPT, RPT_TAIL)])

        plsc.subcore_barrier()

        nch = 39 + jnp.where(wid < (NCHUNKS - 39 * NW), 1, 0)

        @pl.loop(0, nch)
        def _(kk):
            cbase = (wid + NW * kk) * CH
            pltpu.sync_copy(sidx_hbm.at[pl.ds(cbase, CH)], idx_v)
            pltpu.sync_copy(sout_hbm.at[pl.ds(cbase, CH)], rows_v)
            pltpu.sync_copy(rows_v, pooled_sh.at[idx_v], add=True)
            for j in range(CH // 16):
                plsc.addupdate_scatter(cnt_v, [idx_v[pl.ds(j * 16, 16)]], ones16)
            pltpu.sync_copy(oidx_hbm.at[pl.ds(cbase, CH)], idx_v)
            pltpu.sync_copy(oout_hbm.at[pl.ds(cbase, CH)], rows_v)
            pltpu.sync_copy(rows_v, pooled_sh.at[idx_v], add=True)
            for j in range(CH // 16):
                plsc.addupdate_scatter(cnt_v, [idx_v[pl.ds(j * 16, 16)]], ones16)

        pltpu.sync_copy(cnt_v, counts_out.at[wid])
        plsc.subcore_barrier()
        for j in range(4):
            pltpu.sync_copy(pooled_sh.at[pl.ds(base + j * CH, CH)], rows_v)
            pltpu.sync_copy(rows_v, pooled_out.at[cid, pl.ds(base + j * CH, CH)])
        pltpu.sync_copy(pooled_sh.at[pl.ds(base + 4 * CH, RPT - 4 * CH)],
                        rows_v.at[pl.ds(0, RPT - 4 * CH)])
        pltpu.sync_copy(rows_v.at[pl.ds(0, RPT - 4 * CH)],
                        pooled_out.at[cid, pl.ds(base + 4 * CH, RPT - 4 * CH)])

        @pl.when(sid == NS - 1)
        def _():
            pltpu.sync_copy(pooled_sh.at[pl.ds(NS * RPT, RPT_TAIL)],
                            rows_v.at[pl.ds(0, RPT_TAIL)])
            pltpu.sync_copy(rows_v.at[pl.ds(0, RPT_TAIL)],
                            pooled_out.at[cid, pl.ds(NS * RPT, RPT_TAIL)])

    return k(s_out, o_out, s_idx, o_idx, zeros_p)


BT = 2000


def _mlp_body(sg_ref, p_ref, og_ref, w1a_ref, b1a_ref, w1b_ref, b1b_ref,
              s_ref, p_out_ref, o_ref):
    x = jnp.concatenate([sg_ref[...], p_ref[...], og_ref[...]], axis=1)
    h = jnp.maximum(
        jnp.dot(x, w1a_ref[...], preferred_element_type=_f32,
                precision=lax.Precision.HIGHEST) + b1a_ref[...], 0.0)
    t = jnp.maximum(
        jnp.dot(h, w1b_ref[...], preferred_element_type=_f32,
                precision=lax.Precision.HIGHEST) + b1b_ref[...], 0.0)
    s_ref[...] = t[:, :H]
    p_out_ref[...] = t[:, H:H + DOUT]
    o_ref[...] = t[:, H + DOUT:]


def _tc_mlp(sg, p, og, W1a, b1a, W1b, b1b):
    n = T // BT
    row_spec = pl.BlockSpec((BT, DIN), lambda i: (i, 0))
    full = lambda shape: pl.BlockSpec(shape, lambda i: tuple(0 for _ in shape))
    return pl.pallas_call(
        _mlp_body,
        grid=(n,),
        in_specs=[row_spec, row_spec, row_spec,
                  full((3 * DIN, H)), full((1, H)),
                  full((H, 2 * H + DOUT)), full((1, 2 * H + DOUT))],
        out_specs=[pl.BlockSpec((BT, H), lambda i: (i, 0))] * 3,
        out_shape=[jax.ShapeDtypeStruct((T, H), _f32)] * 3,
    )(sg, p, og, W1a, b1a, W1b, b1b)


BO = 2000


def _final_body(pp_ref, cc_ref, w2a_ref, b2a_ref, w2b_ref, b2b_ref, out_ref):
    pooled = pp_ref[0] + pp_ref[1]
    cnt = jnp.sum(cc_ref[0], axis=0)[:, None]
    cnt = jnp.clip(cnt, 1.0, float(O))
    pooled = pooled / cnt
    h2 = jnp.maximum(
        jnp.dot(pooled, w2a_ref[...], preferred_element_type=_f32,
                precision=lax.Precision.HIGHEST) + b2a_ref[...], 0.0)
    out_ref[...] = jnp.maximum(
        jnp.dot(h2, w2b_ref[...], preferred_element_type=_f32,
                precision=lax.Precision.HIGHEST) + b2b_ref[...], 0.0)


def _tc_final(pp, cc, W2a, b2a, W2b, b2b):
    n = O // BO
    full = lambda shape: pl.BlockSpec(shape, lambda i: tuple(0 for _ in shape))
    return pl.pallas_call(
        _final_body,
        grid=(n,),
        in_specs=[pl.BlockSpec((NC, BO, H), lambda i: (0, i, 0)),
                  pl.BlockSpec((1, NW, BO), lambda i: (i, 0, 0)),
                  full((H, H)), full((1, H)), full((H, DOUT)), full((1, DOUT))],
        out_specs=pl.BlockSpec((BO, DOUT), lambda i: (i, 0)),
        out_shape=jax.ShapeDtypeStruct((O, DOUT), _f32),
    )(pp, cc, W2a, b2a, W2b, b2b)


def kernel(obj_vecs, pred_vecs, edges, W1a, b1a, W1b, b1b, W2a, b2a, W2b, b2b):
    s_idx = edges[:, 0]
    o_idx = edges[:, 1]
    zeros_p = jnp.zeros((CH, H), _f32)
    sg, og = _sc_gather(obj_vecs, s_idx, o_idx)
    s_out, p_out, o_out = _tc_mlp(sg, pred_vecs, og,
                                  W1a, b1a.reshape(1, -1),
                                  W1b, b1b.reshape(1, -1))
    pp, cc = _sc_scatter(s_out, o_out, s_idx, o_idx, zeros_p)
    cc = cc.reshape(NW, O // BO, BO).transpose(1, 0, 2)
    new_obj_vecs = _tc_final(pp, cc, W2a, b2a.reshape(1, -1),
                             W2b, b2b.reshape(1, -1))
    return new_obj_vecs, p_out


# speedup vs baseline: 1.1092x; 1.1092x over previous
import dataclasses
import functools

import jax
import jax.numpy as jnp
from jax import lax
from jax.experimental import pallas as pl
from jax.experimental.pallas import tpu as pltpu
from jax.experimental.pallas import tpu_sc as plsc

O = 10000
T = 160000
DIN = 128
H = 128
DOUT = 128

NC = 2
NS = 16
NW = NC * NS

CH = 128
RPT = 624
RPT_TAIL = O - NS * RPT

NSLICE = 2
TS = T // NSLICE

_f32 = jnp.float32


def _layout_cp():
    cp = pltpu.CompilerParams()
    if "needs_layout_passes" in pltpu.CompilerParams.__dataclass_fields__:
        cp = dataclasses.replace(cp, needs_layout_passes=False)
    return cp


def _nch(wid, n):
    nchunks = n // CH
    base = nchunks // NW
    rem = nchunks - base * NW
    return base + jnp.where(wid < rem, 1, 0)


def _sc_gather(obj_vecs, s_idx, o_idx):
    n = s_idx.shape[0]
    mesh = plsc.VectorSubcoreMesh(core_axis_name="c", subcore_axis_name="s")
    out_type = (jax.ShapeDtypeStruct((n, DIN), _f32),
                jax.ShapeDtypeStruct((n, DIN), _f32))

    @functools.partial(
        pl.kernel, mesh=mesh, out_type=out_type,
        scratch_types=[
            pltpu.VMEM((CH,), jnp.int32),
            pltpu.VMEM((CH,), jnp.int32),
            pltpu.VMEM((CH, DIN), _f32),
            pltpu.VMEM((CH, DIN), _f32),
            pltpu.SemaphoreType.DMA,
            pltpu.SemaphoreType.DMA,
        ],
    )
    def k(obj_hbm, sidx_hbm, oidx_hbm, sg_hbm, og_hbm,
          sidx_v, oidx_v, srows_v, orows_v, sem_s, sem_o):
        wid = lax.axis_index("s") * NC + lax.axis_index("c")

        @pl.loop(0, _nch(wid, n))
        def _(kk):
            cbase = (wid + NW * kk) * CH
            pltpu.sync_copy(sidx_hbm.at[pl.ds(cbase, CH)], sidx_v)
            pltpu.sync_copy(oidx_hbm.at[pl.ds(cbase, CH)], oidx_v)
            cs = pltpu.async_copy(obj_hbm.at[sidx_v], srows_v, sem_s)
            co = pltpu.async_copy(obj_hbm.at[oidx_v], orows_v, sem_o)
            cs.wait()
            co.wait()
            pltpu.sync_copy(srows_v, sg_hbm.at[pl.ds(cbase, CH)])
            pltpu.sync_copy(orows_v, og_hbm.at[pl.ds(cbase, CH)])

    return k(obj_vecs, s_idx, o_idx)


def _sc_scatter(s_out, o_out, s_idx, o_idx, zeros_p):
    n = s_idx.shape[0]
    mesh = plsc.VectorSubcoreMesh(core_axis_name="c", subcore_axis_name="s")
    out_type = (jax.ShapeDtypeStruct((NC, O, H), _f32),
                jax.ShapeDtypeStruct((NW, O), _f32))

    @functools.partial(
        pl.kernel, mesh=mesh, out_type=out_type, compiler_params=_layout_cp(),
        scratch_types=[
            pltpu.VMEM((CH,), jnp.int32),
            pltpu.VMEM((CH, H), _f32),
            pltpu.VMEM((O,), _f32),
            pltpu.VMEM_SHARED((O, H), _f32),
        ],
    )
    def k(sout_hbm, oout_hbm, sidx_hbm, oidx_hbm, zp_hbm,
          pooled_out, counts_out,
          idx_v, rows_v, cnt_v, pooled_sh):
        cid = lax.axis_index("c")
        sid = lax.axis_index("s")
        wid = sid * NC + cid
        base = sid * RPT
        zeros16 = jnp.zeros((16,), _f32)
        ones16 = jnp.ones((16,), _f32)

        @pl.loop(0, O, step=16)
        def _(i):
            cnt_v[pl.ds(i, 16)] = zeros16

        pltpu.sync_copy(zp_hbm, rows_v)
        for j in range(4):
            pltpu.sync_copy(rows_v, pooled_sh.at[pl.ds(base + j * CH, CH)])
        pltpu.sync_copy(rows_v.at[pl.ds(0, RPT - 4 * CH)],
                        pooled_sh.at[pl.ds(base + 4 * CH, RPT - 4 * CH)])

        @pl.when(sid == NS - 1)
        def _():
            pltpu.sync_copy(rows_v.at[pl.ds(0, RPT_TAIL)],
                            pooled_sh.at[pl.ds(NS * RPT, RPT_TAIL)])

        plsc.subcore_barrier()

        @pl.loop(0, _nch(wid, n))
        def _(kk):
            cbase = (wid + NW * kk) * CH
            pltpu.sync_copy(sidx_hbm.at[pl.ds(cbase, CH)], idx_v)
            pltpu.sync_copy(sout_hbm.at[pl.ds(cbase, CH)], rows_v)
            pltpu.sync_copy(rows_v, pooled_sh.at[idx_v], add=True)
            for j in range(CH // 16):
                plsc.addupdate_scatter(cnt_v, [idx_v[pl.ds(j * 16, 16)]], ones16)
            pltpu.sync_copy(oidx_hbm.at[pl.ds(cbase, CH)], idx_v)
            pltpu.sync_copy(oout_hbm.at[pl.ds(cbase, CH)], rows_v)
            pltpu.sync_copy(rows_v, pooled_sh.at[idx_v], add=True)
            for j in range(CH // 16):
                plsc.addupdate_scatter(cnt_v, [idx_v[pl.ds(j * 16, 16)]], ones16)

        pltpu.sync_copy(cnt_v, counts_out.at[wid])
        plsc.subcore_barrier()
        for j in range(4):
            pltpu.sync_copy(pooled_sh.at[pl.ds(base + j * CH, CH)], rows_v)
            pltpu.sync_copy(rows_v, pooled_out.at[cid, pl.ds(base + j * CH, CH)])
        pltpu.sync_copy(pooled_sh.at[pl.ds(base + 4 * CH, RPT - 4 * CH)],
                        rows_v.at[pl.ds(0, RPT - 4 * CH)])
        pltpu.sync_copy(rows_v.at[pl.ds(0, RPT - 4 * CH)],
                        pooled_out.at[cid, pl.ds(base + 4 * CH, RPT - 4 * CH)])

        @pl.when(sid == NS - 1)
        def _():
            pltpu.sync_copy(pooled_sh.at[pl.ds(NS * RPT, RPT_TAIL)],
                            rows_v.at[pl.ds(0, RPT_TAIL)])
            pltpu.sync_copy(rows_v.at[pl.ds(0, RPT_TAIL)],
                            pooled_out.at[cid, pl.ds(NS * RPT, RPT_TAIL)])

    return k(s_out, o_out, s_idx, o_idx, zeros_p)


BT = 2000


def _mlp_body(sg_ref, p_ref, og_ref, w1a_ref, b1a_ref, w1b_ref, b1b_ref,
              s_ref, p_out_ref, o_ref):
    x = jnp.concatenate([sg_ref[...], p_ref[...], og_ref[...]], axis=1)
    h = jnp.maximum(
        jnp.dot(x, w1a_ref[...], preferred_element_type=_f32,
                precision=lax.Precision.HIGHEST) + b1a_ref[...], 0.0)
    t = jnp.maximum(
        jnp.dot(h, w1b_ref[...], preferred_element_type=_f32,
                precision=lax.Precision.HIGHEST) + b1b_ref[...], 0.0)
    s_ref[...] = t[:, :H]
    p_out_ref[...] = t[:, H:H + DOUT]
    o_ref[...] = t[:, H + DOUT:]


def _tc_mlp(sg, p, og, W1a, b1a, W1b, b1b):
    n = sg.shape[0]
    row_spec = pl.BlockSpec((BT, DIN), lambda i: (i, 0))
    full = lambda shape: pl.BlockSpec(shape, lambda i: tuple(0 for _ in shape))
    return pl.pallas_call(
        _mlp_body,
        grid=(n // BT,),
        in_specs=[row_spec, row_spec, row_spec,
                  full((3 * DIN, H)), full((1, H)),
                  full((H, 2 * H + DOUT)), full((1, 2 * H + DOUT))],
        out_specs=[pl.BlockSpec((BT, H), lambda i: (i, 0))] * 3,
        out_shape=[jax.ShapeDtypeStruct((n, H), _f32)] * 3,
    )(sg, p, og, W1a, b1a, W1b, b1b)


BO = 2000


def _final_body(pp_ref, cc_ref, w2a_ref, b2a_ref, w2b_ref, b2b_ref, out_ref):
    pooled = jnp.sum(pp_ref[...], axis=0)
    cnt = jnp.sum(cc_ref[0], axis=0)[:, None]
    cnt = jnp.clip(cnt, 1.0, float(O))
    pooled = pooled / cnt
    h2 = jnp.maximum(
        jnp.dot(pooled, w2a_ref[...], preferred_element_type=_f32,
                precision=lax.Precision.HIGHEST) + b2a_ref[...], 0.0)
    out_ref[...] = jnp.maximum(
        jnp.dot(h2, w2b_ref[...], preferred_element_type=_f32,
                precision=lax.Precision.HIGHEST) + b2b_ref[...], 0.0)


def _tc_final(pp, cc, W2a, b2a, W2b, b2b):
    npart = pp.shape[0]
    ncnt = cc.shape[1]
    full = lambda shape: pl.BlockSpec(shape, lambda i: tuple(0 for _ in shape))
    return pl.pallas_call(
        _final_body,
        grid=(O // BO,),
        in_specs=[pl.BlockSpec((npart, BO, H), lambda i: (0, i, 0)),
                  pl.BlockSpec((1, ncnt, BO), lambda i: (i, 0, 0)),
                  full((H, H)), full((1, H)), full((H, DOUT)), full((1, DOUT))],
        out_specs=pl.BlockSpec((BO, DOUT), lambda i: (i, 0)),
        out_shape=jax.ShapeDtypeStruct((O, DOUT), _f32),
    )(pp, cc, W2a, b2a, W2b, b2b)


def kernel(obj_vecs, pred_vecs, edges, W1a, b1a, W1b, b1b, W2a, b2a, W2b, b2b):
    s_idx = edges[:, 0]
    o_idx = edges[:, 1]
    zeros_p = jnp.zeros((CH, H), _f32)

    pps, ccs, pouts = [], [], []
    for k in range(NSLICE):
        lo, hi = k * TS, (k + 1) * TS
        s_k, o_k = s_idx[lo:hi], o_idx[lo:hi]
        sg, og = _sc_gather(obj_vecs, s_k, o_k)
        s_out, p_out, o_out = _tc_mlp(sg, pred_vecs[lo:hi], og,
                                      W1a, b1a.reshape(1, -1),
                                      W1b, b1b.reshape(1, -1))
        pp, cc = _sc_scatter(s_out, o_out, s_k, o_k, zeros_p)
        pps.append(pp)
        ccs.append(cc)
        pouts.append(p_out)

    pp = jnp.concatenate(pps, axis=0)
    cc = jnp.concatenate(ccs, axis=0)
    cc = cc.reshape(NSLICE * NW, O // BO, BO).transpose(1, 0, 2)
    new_p_vecs = jnp.concatenate(pouts, axis=0)
    new_obj_vecs = _tc_final(pp, cc, W2a, b2a.reshape(1, -1),
                             W2b, b2b.reshape(1, -1))
    return new_obj_vecs, new_p_vecs
